# baseline (speedup 1.0000x reference)
import functools

import jax
import jax.numpy as jnp
from jax import lax
from jax.experimental import pallas as pl
from jax.experimental.pallas import tpu as pltpu
from jax.experimental.pallas import tpu_sc as plsc

N = 10000
E = 320000
D = 128
NRBF = 32
NV = 3
B = 100
ZV = 100
CUT = 5.0

_NC = 2
_NS = 16
_NW = _NC * _NS
_EW = E // _NW
_C = 40
_NCHUNK = _EW // _C
_NB = (_NCHUNK - 2) // 4
_NP = 10240
_RPT = _NP // _NS

_BLK = 400
_NBLK = N // _BLK
_EBLK = 2560
_NEBLK = E // _EBLK
_BP = 128


def _sqdist_body(px_hbm, py_hbm, pz_hbm, src_hbm, dst_hbm, out_hbm,
                 px_v, py_v, pz_v, src_v, dst_v, out_v):
    cid = lax.axis_index("c")
    sid = lax.axis_index("s")
    wid = cid * _NS + sid
    pltpu.sync_copy(px_hbm, px_v)
    pltpu.sync_copy(py_hbm, py_v)
    pltpu.sync_copy(pz_hbm, pz_v)
    pltpu.sync_copy(src_hbm.at[pl.ds(wid * _EW, _EW)], src_v)
    pltpu.sync_copy(dst_hbm.at[pl.ds(wid * _EW, _EW)], dst_v)

    def step(j, carry):
        sl = pl.ds(j * 16, 16)
        si = src_v[sl]
        di = dst_v[sl]
        dx = plsc.load_gather(px_v, [si]) - plsc.load_gather(px_v, [di])
        dy = plsc.load_gather(py_v, [si]) - plsc.load_gather(py_v, [di])
        dz = plsc.load_gather(pz_v, [si]) - plsc.load_gather(pz_v, [di])
        out_v[sl] = dx * dx + dy * dy + dz * dz
        return carry

    lax.fori_loop(0, _EW // 16, step, 0, unroll=4)
    pltpu.sync_copy(out_v, out_hbm.at[pl.ds(wid * _EW, _EW)])


@jax.jit
def _sqdist(px, py, pz, src, dst):
    mesh = plsc.VectorSubcoreMesh(core_axis_name="c", subcore_axis_name="s")
    return pl.kernel(
        _sqdist_body,
        mesh=mesh,
        compiler_params=pltpu.CompilerParams(needs_layout_passes=False),
        out_type=jax.ShapeDtypeStruct((E,), jnp.float32),
        scratch_types=[
            pltpu.VMEM((N,), jnp.float32),
            pltpu.VMEM((N,), jnp.float32),
            pltpu.VMEM((N,), jnp.float32),
            pltpu.VMEM((_EW,), jnp.int32),
            pltpu.VMEM((_EW,), jnp.int32),
            pltpu.VMEM((_EW,), jnp.float32),
        ],
    )(px, py, pz, src, dst)


def _edge_mp_body(h_hbm, efilt_hbm, idx_hbm, zeros_hbm, out_hbm,
                  idx_t, rows0, rows1, filt0, filt1, prod0, prod1,
                  agg_sh, gs0, gs1, fs0, fs1, ss0, ss1, is0, is1, is2, is3):
    cid = lax.axis_index("c")
    sid = lax.axis_index("s")
    wid = cid * _NS + sid
    row_base = wid * _EW

    R = (rows0, rows1)
    F = (filt0, filt1)
    P = (prod0, prod1)
    GS = (gs0, gs1)
    FS = (fs0, fs1)
    SS = (ss0, ss1)
    IS = (is0, is1, is2, is3)

    def idx_start(j, q):
        pltpu.async_copy(idx_hbm.at[wid, j], idx_t.at[q], IS[q])

    def idx_wait(j, q):
        pltpu.make_async_copy(idx_hbm.at[wid, j], idx_t.at[q], IS[q]).wait()

    def fetch_start(j, b, q):
        pltpu.async_copy(h_hbm.at[idx_t.at[q, 0]], R[b], GS[b])
        pltpu.async_copy(
            efilt_hbm.at[pl.ds(row_base + j * _C, _C)], F[b], FS[b])

    def fetch_wait(j, b, q):
        pltpu.make_async_copy(h_hbm.at[idx_t.at[q, 0]], R[b], GS[b]).wait()
        pltpu.make_async_copy(
            efilt_hbm.at[pl.ds(row_base + j * _C, _C)], F[b], FS[b]).wait()

    def mul(b):
        rows, filt, prod = R[b], F[b], P[b]

        def mul_row(k, carry2):
            for d in range(D // 16):
                sl = pl.ds(d * 16, 16)
                prod[k, sl] = rows[k, sl] * filt[k, sl]
            return carry2
        lax.fori_loop(0, _C, mul_row, 0, unroll=8)

    def scat_start(b, q):
        pltpu.async_copy(P[b], agg_sh.at[idx_t.at[q, 1]], SS[b], add=True)

    def scat_wait(b, q):
        pltpu.make_async_copy(P[b], agg_sh.at[idx_t.at[q, 1]], SS[b]).wait()

    pltpu.sync_copy(zeros_hbm.at[pl.ds(sid * _RPT, _RPT)],
                    agg_sh.at[pl.ds(sid * _RPT, _RPT)])
    plsc.subcore_barrier()

    idx_start(0, 0)
    idx_start(1, 1)
    idx_start(2, 2)
    idx_wait(0, 0)
    fetch_start(0, 0, 0)
    fetch_wait(0, 0, 0)
    idx_start(3, 3)
    mul(0)
    scat_start(0, 0)
    idx_wait(1, 1)
    fetch_start(1, 1, 1)
    fetch_wait(1, 1, 1)
    mul(1)
    scat_start(1, 1)
    idx_wait(2, 2)
    fetch_start(2, 0, 2)

    def block(k, carry):
        j0 = 4 * k + 2
        for i, (b, q) in enumerate(((0, 2), (1, 3), (0, 0), (1, 1))):
            j = j0 + i
            qf = (q + 2) % 4
            q1 = (q + 1) % 4
            fetch_wait(j, b, q)
            scat_wait(b, qf)
            jn = j + 2
            jn = jnp.where(jn >= _NCHUNK, jn - _NCHUNK, jn)
            idx_start(jn, qf)
            mul(b)
            scat_start(b, q)
            jg = j + 1
            jg = jnp.where(jg >= _NCHUNK, jg - _NCHUNK, jg)
            idx_wait(jg, q1)
            fetch_start(jg, 1 - b, q1)
        return carry

    lax.fori_loop(0, _NB, block, 0)

    scat_wait(0, 0)
    scat_wait(1, 1)
    fetch_wait(0, 0, 2)
    idx_wait(1, 3)

    plsc.subcore_barrier()
    pltpu.sync_copy(agg_sh.at[pl.ds(sid * _RPT, _RPT)],
                    out_hbm.at[cid, pl.ds(sid * _RPT, _RPT)])


@jax.jit
def _edge_mp(h, efilt, idx2, zeros):
    mesh = plsc.VectorSubcoreMesh(core_axis_name="c", subcore_axis_name="s")
    return pl.kernel(
        _edge_mp_body,
        mesh=mesh,
        out_type=jax.ShapeDtypeStruct((_NC, _NP, D), jnp.float32),
        scratch_types=[
            pltpu.VMEM((4, 2, _C), jnp.int32),
            pltpu.VMEM((_C, D), jnp.float32),
            pltpu.VMEM((_C, D), jnp.float32),
            pltpu.VMEM((_C, D), jnp.float32),
            pltpu.VMEM((_C, D), jnp.float32),
            pltpu.VMEM((_C, D), jnp.float32),
            pltpu.VMEM((_C, D), jnp.float32),
            pltpu.VMEM_SHARED((_NP, D), jnp.float32),
            pltpu.SemaphoreType.DMA,
            pltpu.SemaphoreType.DMA,
            pltpu.SemaphoreType.DMA,
            pltpu.SemaphoreType.DMA,
            pltpu.SemaphoreType.DMA,
            pltpu.SemaphoreType.DMA,
            pltpu.SemaphoreType.DMA,
            pltpu.SemaphoreType.DMA,
            pltpu.SemaphoreType.DMA,
            pltpu.SemaphoreType.DMA,
        ],
    )(h, efilt, idx2, zeros)


def _ohT(idx_row, width):
    ids = jax.lax.broadcasted_iota(jnp.int32, (width, idx_row.shape[-1]), 0)
    return (idx_row == ids).astype(jnp.float32)


def _dot0(a, b):
    return jax.lax.dot_general(a, b, (((0,), (0,)), ((), ())),
                               preferred_element_type=jnp.float32)


def _counts_body(batch_ref, out_ref):
    i = pl.program_id(0)

    @pl.when(i == 0)
    def _():
        out_ref[...] = jnp.zeros_like(out_ref)

    ohT = _ohT(batch_ref[0], _BP)
    out_ref[...] += jnp.sum(ohT, axis=1, keepdims=True)


@jax.jit
def _counts_k(batch3):
    return pl.pallas_call(
        _counts_body,
        grid=(_NBLK,),
        in_specs=[pl.BlockSpec((1, 1, _BLK), lambda i: (i, 0, 0))],
        out_specs=pl.BlockSpec((_BP, 1), lambda i: (0, 0)),
        out_shape=jax.ShapeDtypeStruct((_BP, 1), jnp.float32),
    )(batch3)


def _nodes_body(z_ref, batch_ref, counts_ref, emb_ref, wv_ref, wb_ref,
                h_ref, fv_ref, fb_ref):
    recip = 1.0 / jnp.maximum(counts_ref[...], 1.0)
    ohbT = _ohT(batch_ref[0], _BP)
    w = _dot0(recip, ohbT)
    cw = jax.lax.broadcasted_iota(jnp.int32, (NRBF, 1), 0).astype(jnp.float32) * (1.0 / (NRBF - 1))
    rbfT = jnp.exp(-50.0 * (w - cw) ** 2)
    fv_ref[...] = _dot0(rbfT, wv_ref[...])
    fb_ref[...] = _dot0(rbfT, wb_ref[...])
    ohzT = _ohT(z_ref[0], ZV)
    h_ref[...] = _dot0(ohzT, emb_ref[...])


@jax.jit
def _nodes_k(z3, batch3, counts, emb, W_vrbf, W_brbf):
    out = jax.ShapeDtypeStruct((N, D), jnp.float32)
    return pl.pallas_call(
        _nodes_body,
        grid=(_NBLK,),
        in_specs=[
            pl.BlockSpec((1, 1, _BLK), lambda i: (i, 0, 0)),
            pl.BlockSpec((1, 1, _BLK), lambda i: (i, 0, 0)),
            pl.BlockSpec((_BP, 1), lambda i: (0, 0)),
            pl.BlockSpec((ZV, D), lambda i: (0, 0)),
            pl.BlockSpec((NRBF, D), lambda i: (0, 0)),
            pl.BlockSpec((NRBF, D), lambda i: (0, 0)),
        ],
        out_specs=[
            pl.BlockSpec((_BLK, D), lambda i: (i, 0)),
            pl.BlockSpec((_BLK, D), lambda i: (i, 0)),
            pl.BlockSpec((_BLK, D), lambda i: (i, 0)),
        ],
        out_shape=[out, out, out],
    )(z3, batch3, counts, emb, W_vrbf, W_brbf)


def _efilt_body(sq_ref, wr_ref, out_ref):
    dist = jnp.sqrt(sq_ref[0] + 1e-9)
    ce = jax.lax.broadcasted_iota(jnp.int32, (NRBF, 1), 0).astype(jnp.float32) * (CUT / (NRBF - 1))
    erbfT = jnp.exp(-10.0 * (dist - ce) ** 2)
    env = 0.5 * (jnp.cos(jnp.pi * jnp.clip(dist * (1.0 / CUT), 0.0, 1.0)) + 1.0)
    out_ref[...] = _dot0(erbfT * env, wr_ref[...])


@jax.jit
def _efilt_k(sq3, W_rbf):
    return pl.pallas_call(
        _efilt_body,
        grid=(_NEBLK,),
        in_specs=[
            pl.BlockSpec((1, 1, _EBLK), lambda i: (i, 0, 0)),
            pl.BlockSpec((NRBF, D), lambda i: (0, 0)),
        ],
        out_specs=pl.BlockSpec((_EBLK, D), lambda i: (i, 0)),
        out_shape=jax.ShapeDtypeStruct((E, D), jnp.float32),
    )(sq3, W_rbf)


def _S_body(batch_ref, h_ref, fv_ref, s0_ref, out_ref):
    i = pl.program_id(0)

    @pl.when(i == 0)
    def _():
        out_ref[...] = s0_ref[...]

    ohbT = _ohT(batch_ref[0], _BP)
    hv = h_ref[...] * fv_ref[...]
    out_ref[...] += float(NV) * jnp.dot(
        ohbT, hv, preferred_element_type=jnp.float32)


@jax.jit
def _S_k(batch3, h, F_v, S0):
    return pl.pallas_call(
        _S_body,
        grid=(_NBLK,),
        in_specs=[
            pl.BlockSpec((1, 1, _BLK), lambda i: (i, 0, 0)),
            pl.BlockSpec((_BLK, D), lambda i: (i, 0)),
            pl.BlockSpec((_BLK, D), lambda i: (i, 0)),
            pl.BlockSpec((_BP, D), lambda i: (0, 0)),
        ],
        out_specs=pl.BlockSpec((_BP, D), lambda i: (0, 0)),
        out_shape=jax.ShapeDtypeStruct((_BP, D), jnp.float32),
    )(batch3, h, F_v, S0)


def _hup_body(batch_ref, h_ref, agg_ref, fb_ref, s_ref, w1_ref, w2_ref,
              out_ref):
    ohbT = _ohT(batch_ref[0], _BP)
    bcast = fb_ref[...] * _dot0(ohbT, s_ref[...])
    a = agg_ref[0] + agg_ref[1] + bcast
    t = jnp.dot(a, w1_ref[...], preferred_element_type=jnp.float32)
    t = t * jax.nn.sigmoid(t)
    out_ref[...] = h_ref[...] + jnp.dot(t, w2_ref[...],
                                        preferred_element_type=jnp.float32)


@jax.jit
def _hup_k(batch3, h, parts, F_b, S, W1i, W2i):
    return pl.pallas_call(
        _hup_body,
        grid=(_NBLK,),
        in_specs=[
            pl.BlockSpec((1, 1, _BLK), lambda i: (i, 0, 0)),
            pl.BlockSpec((_BLK, D), lambda i: (i, 0)),
            pl.BlockSpec((2, _BLK, D), lambda i: (0, i, 0)),
            pl.BlockSpec((_BLK, D), lambda i: (i, 0)),
            pl.BlockSpec((_BP, D), lambda i: (0, 0)),
            pl.BlockSpec((D, D), lambda i: (0, 0)),
            pl.BlockSpec((D, D), lambda i: (0, 0)),
        ],
        out_specs=pl.BlockSpec((_BLK, D), lambda i: (i, 0)),
        out_shape=jax.ShapeDtypeStruct((N, D), jnp.float32),
    )(batch3, h, parts, F_b, S, W1i, W2i)


def _out_body(batch_ref, h_ref, lng_ref, lnb_ref, wo1_ref, bo1_ref,
              wo2_ref, bo2_ref, out_ref):
    i = pl.program_id(0)

    @pl.when(i == 0)
    def _():
        out_ref[...] = jnp.zeros_like(out_ref)

    h = h_ref[...]
    mu = jnp.mean(h, axis=-1, keepdims=True)
    xc = h - mu
    var = jnp.mean(xc * xc, axis=-1, keepdims=True)
    hn = xc * jax.lax.rsqrt(var + 1e-5) * lng_ref[...] + lnb_ref[...]
    t = jnp.dot(hn, wo1_ref[...], preferred_element_type=jnp.float32) + bo1_ref[...]
    t = t * jax.nn.sigmoid(t)
    e = jnp.dot(t, wo2_ref[...], preferred_element_type=jnp.float32) + bo2_ref[...]
    ohbT = _ohT(batch_ref[0], _BP)
    out_ref[...] += jnp.dot(ohbT, e, preferred_element_type=jnp.float32)


@jax.jit
def _out_k(batch3, h, ln_g, ln_b, Wo1, bo1, Wo2, bo2):
    return pl.pallas_call(
        _out_body,
        grid=(_NBLK,),
        in_specs=[
            pl.BlockSpec((1, 1, _BLK), lambda i: (i, 0, 0)),
            pl.BlockSpec((_BLK, D), lambda i: (i, 0)),
            pl.BlockSpec((1, D), lambda i: (0, 0)),
            pl.BlockSpec((1, D), lambda i: (0, 0)),
            pl.BlockSpec((D, D // 2), lambda i: (0, 0)),
            pl.BlockSpec((1, D // 2), lambda i: (0, 0)),
            pl.BlockSpec((D // 2, 1), lambda i: (0, 0)),
            pl.BlockSpec((1, 1), lambda i: (0, 0)),
        ],
        out_specs=pl.BlockSpec((_BP, 1), lambda i: (0, 0)),
        out_shape=jax.ShapeDtypeStruct((_BP, 1), jnp.float32),
    )(batch3, h, ln_g, ln_b, Wo1, bo1, Wo2, bo2)


def kernel(z, pos, edge_index, batch, emb, virt_emb, W_rbf, W_vrbf, W_brbf,
           W1, W2, ln_g, ln_b, Wo1, bo1, Wo2, bo2):
    src, dst = edge_index[0], edge_index[1]
    order = jnp.argsort(dst)
    src = src[order]
    dst = dst[order]
    idx2 = jnp.stack([src.reshape(_NW, _NCHUNK, _C),
                      dst.reshape(_NW, _NCHUNK, _C)], axis=2)
    zeros = jnp.zeros((_NP, D), jnp.float32)
    z3 = z.reshape(_NBLK, 1, _BLK).astype(jnp.int32)
    batch3 = batch.reshape(_NBLK, 1, _BLK).astype(jnp.int32)
    posT = pos.T

    sq = _sqdist(posT[0], posT[1], posT[2], src, dst)
    efilt = _efilt_k(sq.reshape(_NEBLK, 1, _EBLK), W_rbf)

    counts = _counts_k(batch3)
    h, F_v, F_b = _nodes_k(z3, batch3, counts, emb, W_vrbf, W_brbf)

    S = jnp.zeros((_BP, D), jnp.float32).at[:B].set(
        jnp.broadcast_to(jnp.sum(virt_emb, axis=0), (B, D)))
    for i in range(3):
        parts = _edge_mp(h, efilt, idx2, zeros)
        parts_n = jnp.stack([parts[0, :N], parts[1, :N]])
        S = _S_k(batch3, h, F_v, S)
        h = _hup_k(batch3, h, parts_n, F_b, S, W1[i], W2[i])

    energy = _out_k(batch3, h, ln_g.reshape(1, D), ln_b.reshape(1, D),
                    Wo1, bo1.reshape(1, D // 2), Wo2, bo2.reshape(1, 1))
    return energy[:B, 0]


# speedup vs baseline: 1.5810x; 1.5810x over previous
import functools

import jax
import jax.numpy as jnp
from jax import lax
from jax.experimental import pallas as pl
from jax.experimental.pallas import tpu as pltpu
from jax.experimental.pallas import tpu_sc as plsc

N = 10000
E = 320000
D = 128
NRBF = 32
NV = 3
B = 100
ZV = 100
CUT = 5.0

_NC = 2
_NS = 16
_NW = _NC * _NS
_EW = E // _NW
_C = 40
_NCHUNK = _EW // _C
_NB = (_NCHUNK - 2) // 4
_NP = 10240
_RPT = _NP // _NS

_BLK = 400
_NBLK = N // _BLK
_EBLK = 2560
_NEBLK = E // _EBLK
_BP = 128


def _sqdist_body(px_hbm, py_hbm, pz_hbm, src_hbm, dst_hbm, out_hbm,
                 px_v, py_v, pz_v, src_v, dst_v, out_v):
    cid = lax.axis_index("c")
    sid = lax.axis_index("s")
    wid = cid * _NS + sid
    pltpu.sync_copy(px_hbm, px_v)
    pltpu.sync_copy(py_hbm, py_v)
    pltpu.sync_copy(pz_hbm, pz_v)
    pltpu.sync_copy(src_hbm.at[pl.ds(wid * _EW, _EW)], src_v)
    pltpu.sync_copy(dst_hbm.at[pl.ds(wid * _EW, _EW)], dst_v)

    def step(j, carry):
        sl = pl.ds(j * 16, 16)
        si = src_v[sl]
        di = dst_v[sl]
        dx = plsc.load_gather(px_v, [si]) - plsc.load_gather(px_v, [di])
        dy = plsc.load_gather(py_v, [si]) - plsc.load_gather(py_v, [di])
        dz = plsc.load_gather(pz_v, [si]) - plsc.load_gather(pz_v, [di])
        out_v[sl] = dx * dx + dy * dy + dz * dz
        return carry

    lax.fori_loop(0, _EW // 16, step, 0, unroll=4)
    pltpu.sync_copy(out_v, out_hbm.at[pl.ds(wid * _EW, _EW)])


@jax.jit
def _sqdist(px, py, pz, src, dst):
    mesh = plsc.VectorSubcoreMesh(core_axis_name="c", subcore_axis_name="s")
    return pl.kernel(
        _sqdist_body,
        mesh=mesh,
        compiler_params=pltpu.CompilerParams(needs_layout_passes=False),
        out_type=jax.ShapeDtypeStruct((E,), jnp.float32),
        scratch_types=[
            pltpu.VMEM((N,), jnp.float32),
            pltpu.VMEM((N,), jnp.float32),
            pltpu.VMEM((N,), jnp.float32),
            pltpu.VMEM((_EW,), jnp.int32),
            pltpu.VMEM((_EW,), jnp.int32),
            pltpu.VMEM((_EW,), jnp.float32),
        ],
    )(px, py, pz, src, dst)


def _edge_mp_body(h_hbm, efilt_hbm, src_hbm, dst_hbm, zeros_hbm, out_hbm,
                  src_t, dst_t, rows0, rows1, filt0, filt1, prod0, prod1,
                  agg_sh, gs0, gs1, fs0, fs1, ss0, ss1, is0, is1, is2, is3):
    cid = lax.axis_index("c")
    sid = lax.axis_index("s")
    wid = cid * _NS + sid
    row_base = wid * _EW

    R = (rows0, rows1)
    F = (filt0, filt1)
    P = (prod0, prod1)
    GS = (gs0, gs1)
    FS = (fs0, fs1)
    SS = (ss0, ss1)
    IS = (is0, is1, is2, is3)

    def idx_start(j, q):
        pltpu.async_copy(src_hbm.at[wid, j], src_t.at[q], IS[q])
        pltpu.async_copy(dst_hbm.at[wid, j], dst_t.at[q], IS[q])

    def idx_wait(j, q):
        pltpu.make_async_copy(src_hbm.at[wid, j], src_t.at[q], IS[q]).wait()
        pltpu.make_async_copy(dst_hbm.at[wid, j], dst_t.at[q], IS[q]).wait()

    def fetch_start(j, b, q):
        pltpu.async_copy(h_hbm.at[src_t.at[q]], R[b], GS[b])
        pltpu.async_copy(
            efilt_hbm.at[pl.ds(row_base + j * _C, _C)], F[b], FS[b])

    def fetch_wait(j, b, q):
        pltpu.make_async_copy(h_hbm.at[src_t.at[q]], R[b], GS[b]).wait()
        pltpu.make_async_copy(
            efilt_hbm.at[pl.ds(row_base + j * _C, _C)], F[b], FS[b]).wait()

    def mul(b):
        rows, filt, prod = R[b], F[b], P[b]

        def mul_row(k, carry2):
            for d in range(D // 16):
                sl = pl.ds(d * 16, 16)
                prod[k, sl] = rows[k, sl] * filt[k, sl]
            return carry2
        lax.fori_loop(0, _C, mul_row, 0, unroll=4)

    def scat_start(b, q):
        pltpu.async_copy(P[b], agg_sh.at[dst_t.at[q]], SS[b], add=True)

    def scat_wait(b, q):
        pltpu.make_async_copy(P[b], agg_sh.at[dst_t.at[q]], SS[b]).wait()

    pltpu.sync_copy(zeros_hbm.at[pl.ds(sid * _RPT, _RPT)],
                    agg_sh.at[pl.ds(sid * _RPT, _RPT)])
    plsc.subcore_barrier()

    idx_start(0, 0)
    idx_start(1, 1)
    idx_start(2, 2)
    idx_wait(0, 0)
    fetch_start(0, 0, 0)
    fetch_wait(0, 0, 0)
    idx_wait(1, 1)
    fetch_start(1, 1, 1)
    idx_start(3, 3)
    mul(0)
    scat_start(0, 0)
    fetch_wait(1, 1, 1)
    idx_wait(2, 2)
    fetch_start(2, 0, 2)
    mul(1)
    scat_start(1, 1)

    def block(k, carry):
        j0 = 4 * k + 2
        for i, (b, q) in enumerate(((0, 2), (1, 3), (0, 0), (1, 1))):
            j = j0 + i
            qf = (q + 2) % 4
            q1 = (q + 1) % 4
            fetch_wait(j, b, q)
            jg = j + 1
            jg = jnp.where(jg >= _NCHUNK, jg - _NCHUNK, jg)
            idx_wait(jg, q1)
            fetch_start(jg, 1 - b, q1)
            scat_wait(b, qf)
            jn = j + 2
            jn = jnp.where(jn >= _NCHUNK, jn - _NCHUNK, jn)
            idx_start(jn, qf)
            mul(b)
            scat_start(b, q)
        return carry

    lax.fori_loop(0, _NB, block, 0)

    scat_wait(0, 0)
    scat_wait(1, 1)
    fetch_wait(0, 0, 2)
    idx_wait(1, 3)

    plsc.subcore_barrier()
    pltpu.sync_copy(agg_sh.at[pl.ds(sid * _RPT, _RPT)],
                    out_hbm.at[cid, pl.ds(sid * _RPT, _RPT)])


@jax.jit
def _edge_mp(h, efilt, src2, dst2, zeros):
    mesh = plsc.VectorSubcoreMesh(core_axis_name="c", subcore_axis_name="s")
    return pl.kernel(
        _edge_mp_body,
        mesh=mesh,
        out_type=jax.ShapeDtypeStruct((_NC, _NP, D), jnp.float32),
        scratch_types=[
            pltpu.VMEM((4, _C), jnp.int32),
            pltpu.VMEM((4, _C), jnp.int32),
            pltpu.VMEM((_C, D), jnp.float32),
            pltpu.VMEM((_C, D), jnp.float32),
            pltpu.VMEM((_C, D), jnp.float32),
            pltpu.VMEM((_C, D), jnp.float32),
            pltpu.VMEM((_C, D), jnp.float32),
            pltpu.VMEM((_C, D), jnp.float32),
            pltpu.VMEM_SHARED((_NP, D), jnp.float32),
            pltpu.SemaphoreType.DMA,
            pltpu.SemaphoreType.DMA,
            pltpu.SemaphoreType.DMA,
            pltpu.SemaphoreType.DMA,
            pltpu.SemaphoreType.DMA,
            pltpu.SemaphoreType.DMA,
            pltpu.SemaphoreType.DMA,
            pltpu.SemaphoreType.DMA,
            pltpu.SemaphoreType.DMA,
            pltpu.SemaphoreType.DMA,
        ],
    )(h, efilt, src2, dst2, zeros)


def _ohT(idx_row, width):
    ids = jax.lax.broadcasted_iota(jnp.int32, (width, idx_row.shape[-1]), 0)
    return (idx_row == ids).astype(jnp.float32)


def _dot0(a, b):
    return jax.lax.dot_general(a, b, (((0,), (0,)), ((), ())),
                               precision=jax.lax.Precision.HIGHEST,
                               preferred_element_type=jnp.float32)


def _counts_body(batch_ref, out_ref):
    i = pl.program_id(0)

    @pl.when(i == 0)
    def _():
        out_ref[...] = jnp.zeros_like(out_ref)

    ohT = _ohT(batch_ref[0], _BP)
    out_ref[...] += jnp.sum(ohT, axis=1, keepdims=True)


@jax.jit
def _counts_k(batch3):
    return pl.pallas_call(
        _counts_body,
        grid=(_NBLK,),
        in_specs=[pl.BlockSpec((1, 1, _BLK), lambda i: (i, 0, 0))],
        out_specs=pl.BlockSpec((_BP, 1), lambda i: (0, 0)),
        out_shape=jax.ShapeDtypeStruct((_BP, 1), jnp.float32),
    )(batch3)


def _nodes_body(z_ref, batch_ref, counts_ref, emb_ref, wv_ref, wb_ref,
                h_ref, fv_ref, fb_ref):
    recip = 1.0 / jnp.maximum(counts_ref[...], 1.0)
    ohbT = _ohT(batch_ref[0], _BP)
    w = _dot0(recip, ohbT)
    cw = jax.lax.broadcasted_iota(jnp.int32, (NRBF, 1), 0).astype(jnp.float32) * (1.0 / (NRBF - 1))
    rbfT = jnp.exp(-50.0 * (w - cw) ** 2)
    fv_ref[...] = _dot0(rbfT, wv_ref[...])
    fb_ref[...] = _dot0(rbfT, wb_ref[...])
    ohzT = _ohT(z_ref[0], ZV)
    h_ref[...] = _dot0(ohzT, emb_ref[...])


@jax.jit
def _nodes_k(z3, batch3, counts, emb, W_vrbf, W_brbf):
    out = jax.ShapeDtypeStruct((N, D), jnp.float32)
    return pl.pallas_call(
        _nodes_body,
        grid=(_NBLK,),
        in_specs=[
            pl.BlockSpec((1, 1, _BLK), lambda i: (i, 0, 0)),
            pl.BlockSpec((1, 1, _BLK), lambda i: (i, 0, 0)),
            pl.BlockSpec((_BP, 1), lambda i: (0, 0)),
            pl.BlockSpec((ZV, D), lambda i: (0, 0)),
            pl.BlockSpec((NRBF, D), lambda i: (0, 0)),
            pl.BlockSpec((NRBF, D), lambda i: (0, 0)),
        ],
        out_specs=[
            pl.BlockSpec((_BLK, D), lambda i: (i, 0)),
            pl.BlockSpec((_BLK, D), lambda i: (i, 0)),
            pl.BlockSpec((_BLK, D), lambda i: (i, 0)),
        ],
        out_shape=[out, out, out],
    )(z3, batch3, counts, emb, W_vrbf, W_brbf)


def _efilt_body(sq_ref, wr_ref, out_ref):
    dist = jnp.sqrt(sq_ref[0] + 1e-9)
    ce = jax.lax.broadcasted_iota(jnp.int32, (NRBF, 1), 0).astype(jnp.float32) * (CUT / (NRBF - 1))
    erbfT = jnp.exp(-10.0 * (dist - ce) ** 2)
    env = 0.5 * (jnp.cos(jnp.pi * jnp.clip(dist * (1.0 / CUT), 0.0, 1.0)) + 1.0)
    out_ref[...] = _dot0(erbfT * env, wr_ref[...])


@jax.jit
def _efilt_k(sq3, W_rbf):
    return pl.pallas_call(
        _efilt_body,
        grid=(_NEBLK,),
        in_specs=[
            pl.BlockSpec((1, 1, _EBLK), lambda i: (i, 0, 0)),
            pl.BlockSpec((NRBF, D), lambda i: (0, 0)),
        ],
        out_specs=pl.BlockSpec((_EBLK, D), lambda i: (i, 0)),
        out_shape=jax.ShapeDtypeStruct((E, D), jnp.float32),
    )(sq3, W_rbf)


def _S_body(batch_ref, h_ref, fv_ref, s0_ref, out_ref):
    i = pl.program_id(0)

    @pl.when(i == 0)
    def _():
        out_ref[...] = s0_ref[...]

    ohbT = _ohT(batch_ref[0], _BP)
    hv = h_ref[...] * fv_ref[...]
    out_ref[...] += float(NV) * jnp.dot(
        ohbT, hv, precision=jax.lax.Precision.HIGHEST,
        preferred_element_type=jnp.float32)


@jax.jit
def _S_k(batch3, h, F_v, S0):
    return pl.pallas_call(
        _S_body,
        grid=(_NBLK,),
        in_specs=[
            pl.BlockSpec((1, 1, _BLK), lambda i: (i, 0, 0)),
            pl.BlockSpec((_BLK, D), lambda i: (i, 0)),
            pl.BlockSpec((_BLK, D), lambda i: (i, 0)),
            pl.BlockSpec((_BP, D), lambda i: (0, 0)),
        ],
        out_specs=pl.BlockSpec((_BP, D), lambda i: (0, 0)),
        out_shape=jax.ShapeDtypeStruct((_BP, D), jnp.float32),
    )(batch3, h, F_v, S0)


def _hup_body(batch_ref, h_ref, agg_ref, fb_ref, s_ref, w1_ref, w2_ref,
              out_ref):
    ohbT = _ohT(batch_ref[0], _BP)
    bcast = fb_ref[...] * _dot0(ohbT, s_ref[...])
    a = agg_ref[0] + agg_ref[1] + bcast
    t = jnp.dot(a, w1_ref[...], precision=jax.lax.Precision.HIGHEST, preferred_element_type=jnp.float32)
    t = t * jax.nn.sigmoid(t)
    out_ref[...] = h_ref[...] + jnp.dot(t, w2_ref[...],
                                        precision=jax.lax.Precision.HIGHEST, preferred_element_type=jnp.float32)


@jax.jit
def _hup_k(batch3, h, parts, F_b, S, W1i, W2i):
    return pl.pallas_call(
        _hup_body,
        grid=(_NBLK,),
        in_specs=[
            pl.BlockSpec((1, 1, _BLK), lambda i: (i, 0, 0)),
            pl.BlockSpec((_BLK, D), lambda i: (i, 0)),
            pl.BlockSpec((2, _BLK, D), lambda i: (0, i, 0)),
            pl.BlockSpec((_BLK, D), lambda i: (i, 0)),
            pl.BlockSpec((_BP, D), lambda i: (0, 0)),
            pl.BlockSpec((D, D), lambda i: (0, 0)),
            pl.BlockSpec((D, D), lambda i: (0, 0)),
        ],
        out_specs=pl.BlockSpec((_BLK, D), lambda i: (i, 0)),
        out_shape=jax.ShapeDtypeStruct((N, D), jnp.float32),
    )(batch3, h, parts, F_b, S, W1i, W2i)


def _out_body(batch_ref, h_ref, lng_ref, lnb_ref, wo1_ref, bo1_ref,
              wo2_ref, bo2_ref, out_ref):
    i = pl.program_id(0)

    @pl.when(i == 0)
    def _():
        out_ref[...] = jnp.zeros_like(out_ref)

    h = h_ref[...]
    mu = jnp.mean(h, axis=-1, keepdims=True)
    xc = h - mu
    var = jnp.mean(xc * xc, axis=-1, keepdims=True)
    hn = xc * jax.lax.rsqrt(var + 1e-5) * lng_ref[...] + lnb_ref[...]
    t = jnp.dot(hn, wo1_ref[...], precision=jax.lax.Precision.HIGHEST, preferred_element_type=jnp.float32) + bo1_ref[...]
    t = t * jax.nn.sigmoid(t)
    e = jnp.dot(t, wo2_ref[...], precision=jax.lax.Precision.HIGHEST, preferred_element_type=jnp.float32) + bo2_ref[...]
    ohbT = _ohT(batch_ref[0], _BP)
    out_ref[...] += jnp.dot(ohbT, e, precision=jax.lax.Precision.HIGHEST, preferred_element_type=jnp.float32)


@jax.jit
def _out_k(batch3, h, ln_g, ln_b, Wo1, bo1, Wo2, bo2):
    return pl.pallas_call(
        _out_body,
        grid=(_NBLK,),
        in_specs=[
            pl.BlockSpec((1, 1, _BLK), lambda i: (i, 0, 0)),
            pl.BlockSpec((_BLK, D), lambda i: (i, 0)),
            pl.BlockSpec((1, D), lambda i: (0, 0)),
            pl.BlockSpec((1, D), lambda i: (0, 0)),
            pl.BlockSpec((D, D // 2), lambda i: (0, 0)),
            pl.BlockSpec((1, D // 2), lambda i: (0, 0)),
            pl.BlockSpec((D // 2, 1), lambda i: (0, 0)),
            pl.BlockSpec((1, 1), lambda i: (0, 0)),
        ],
        out_specs=pl.BlockSpec((_BP, 1), lambda i: (0, 0)),
        out_shape=jax.ShapeDtypeStruct((_BP, 1), jnp.float32),
    )(batch3, h, ln_g, ln_b, Wo1, bo1, Wo2, bo2)


def kernel(z, pos, edge_index, batch, emb, virt_emb, W_rbf, W_vrbf, W_brbf,
           W1, W2, ln_g, ln_b, Wo1, bo1, Wo2, bo2):
    src, dst = edge_index[0], edge_index[1]
    src2 = src.reshape(_NW, _NCHUNK, _C)
    dst2 = dst.reshape(_NW, _NCHUNK, _C)
    zeros = jnp.zeros((_NP, D), jnp.float32)
    z3 = z.reshape(_NBLK, 1, _BLK).astype(jnp.int32)
    batch3 = batch.reshape(_NBLK, 1, _BLK).astype(jnp.int32)
    posT = pos.T

    sq = _sqdist(posT[0], posT[1], posT[2], src, dst)
    efilt = _efilt_k(sq.reshape(_NEBLK, 1, _EBLK), W_rbf)

    counts = _counts_k(batch3)
    h, F_v, F_b = _nodes_k(z3, batch3, counts, emb, W_vrbf, W_brbf)

    S = jnp.zeros((_BP, D), jnp.float32).at[:B].set(
        jnp.broadcast_to(jnp.sum(virt_emb, axis=0), (B, D)))
    for i in range(3):
        parts = _edge_mp(h, efilt, src2, dst2, zeros)
        parts_n = jnp.stack([parts[0, :N], parts[1, :N]])
        S = _S_k(batch3, h, F_v, S)
        h = _hup_k(batch3, h, parts_n, F_b, S, W1[i], W2[i])

    energy = _out_k(batch3, h, ln_g.reshape(1, D), ln_b.reshape(1, D),
                    Wo1, bo1.reshape(1, D // 2), Wo2, bo2.reshape(1, 1))
    return energy[:B, 0]
